# baseline (speedup 1.0000x reference)
import functools

import jax
import jax.numpy as jnp
from jax import lax
from jax.experimental import pallas as pl
from jax.experimental.pallas import tpu as pltpu
from jax.experimental.pallas import tpu_sc as plsc

_DEPTH = 1000
_ROWS = 16384
_NW = 32
_RW = _ROWS // _NW
_CW = 128
_NCH = _RW // _CW
_VPC = _CW // 16

_mesh = plsc.VectorSubcoreMesh(core_axis_name="c", subcore_axis_name="s")


@functools.partial(
    pl.kernel,
    mesh=_mesh,
    out_type=jax.ShapeDtypeStruct((_DEPTH, _ROWS), jnp.float32),
    scratch_types=[
        pltpu.VMEM((_RW,), jnp.int32),
        pltpu.VMEM((_DEPTH, _CW), jnp.float32),
    ],
)
def _sc_onehot(x_hbm, out_hbm, idx_v, buf):
    wid = lax.axis_index("s") * 2 + lax.axis_index("c")
    wbase = wid * _RW
    pltpu.sync_copy(x_hbm.at[pl.ds(wbase, _RW)], idx_v)

    ones = jnp.ones((16,), jnp.float32)
    zeros = jnp.zeros((16,), jnp.float32)

    def _chunk(c, carry):
        base_r = c * _CW
        ivs = [idx_v[pl.ds(base_r + v * 16, 16)] for v in range(_VPC)]

        def _row(d, carry2):
            dv = jnp.full((16,), d, jnp.int32)
            for v in range(_VPC):
                buf[d, pl.ds(v * 16, 16)] = jnp.where(ivs[v] == dv, ones, zeros)
            return carry2

        lax.fori_loop(0, _DEPTH, _row, 0)
        pltpu.sync_copy(buf, out_hbm.at[:, pl.ds(wbase + base_r, _CW)])
        return carry

    lax.fori_loop(0, _NCH, _chunk, 0)


def sc_onehot(x):
    xi = x.astype(jnp.int32).reshape(_ROWS)
    out = _sc_onehot(xi)
    return out.T.reshape(_ROWS, 1, _DEPTH)


kernel = sc_onehot


# speedup vs baseline: 2.5304x; 2.5304x over previous
import jax
import jax.numpy as jnp
from jax.experimental import pallas as pl

_DEPTH = 1000
_ROWS = 16384
_DBLK = 200
_RBLK = 4096


def _onehot_body(x_ref, o_ref):
    idx = x_ref[...]
    base = pl.program_id(0) * _DBLK
    iota = base + jax.lax.broadcasted_iota(jnp.int32, (_DBLK, _RBLK), 0)
    o_ref[...] = (iota == idx).astype(jnp.float32)


def kernel(x):
    xi = x.astype(jnp.int32).reshape(1, _ROWS)
    out = pl.pallas_call(
        _onehot_body,
        grid=(_DEPTH // _DBLK, _ROWS // _RBLK),
        in_specs=[pl.BlockSpec((1, _RBLK), lambda i, j: (0, j))],
        out_specs=pl.BlockSpec((_DBLK, _RBLK), lambda i, j: (i, j)),
        out_shape=jax.ShapeDtypeStruct((_DEPTH, _ROWS), jnp.float32),
    )(xi)
    return out.T.reshape(_ROWS, 1, _DEPTH)


# speedup vs baseline: 2.7613x; 1.0913x over previous
import jax
import jax.numpy as jnp
from jax.experimental import pallas as pl

_DEPTH = 1000
_ROWS = 16384
_RBLK = 1024


def _onehot_body(x_ref, o_ref):
    idx = x_ref[...]
    iota = jax.lax.broadcasted_iota(jnp.int32, (_DEPTH, _RBLK), 0)
    o_ref[...] = (iota == idx).astype(jnp.float32)


def kernel(x):
    xi = x.astype(jnp.int32).reshape(1, _ROWS)
    out = pl.pallas_call(
        _onehot_body,
        grid=(_ROWS // _RBLK,),
        in_specs=[pl.BlockSpec((1, _RBLK), lambda i: (0, i))],
        out_specs=pl.BlockSpec((_DEPTH, _RBLK), lambda i: (0, i)),
        out_shape=jax.ShapeDtypeStruct((_DEPTH, _ROWS), jnp.float32),
    )(xi)
    return out.T.reshape(_ROWS, 1, _DEPTH)
